# baseline (speedup 1.0000x reference)
import jax
import jax.numpy as jnp
from jax.experimental import pallas as pl
from jax.experimental.pallas import tpu as pltpu

_TILE = 256
_K = 8


def _chamfer_kernel(p_ref, q_ref, o_ref, pb_ref):
    t = pl.program_id(1)

    @pl.when(t == 0)
    def _prologue():
        p = p_ref[0]
        px = p[0:1, :]
        py = p[1:2, :]
        pz = p[2:3, :]
        pn = px * px + py * py + pz * pz
        hi = pn.astype(jnp.bfloat16)
        lo = (pn - hi.astype(jnp.float32)).astype(jnp.bfloat16)
        pb_ref[...] = jnp.concatenate(
            [
                (2.0 * px).astype(jnp.bfloat16),
                (-2.0 * py).astype(jnp.bfloat16),
                (-2.0 * pz).astype(jnp.bfloat16),
                hi,
                lo,
                jnp.zeros((_K - 5, p.shape[1]), jnp.bfloat16),
            ],
            axis=0,
        )
        o_ref[0] = jnp.full((8, 128), jnp.sum(pn), jnp.float32)

    qb = q_ref[0].astype(jnp.bfloat16)
    d = jnp.dot(qb, pb_ref[...], preferred_element_type=jnp.float32)
    s = jnp.sum(jnp.min(d, axis=1))
    o_ref[0] += jnp.full((8, 128), s, jnp.float32)


def kernel(xyz):
    B, N, _ = xyz.shape
    nt = N // _TILE
    qmat = jnp.concatenate(
        [
            xyz,
            jnp.ones((B, N, 2), jnp.float32),
            jnp.zeros((B, N, _K - 5), jnp.float32),
        ],
        axis=2,
    )
    pmat = jnp.swapaxes(xyz, 1, 2)
    out = pl.pallas_call(
        _chamfer_kernel,
        grid=(B, nt),
        in_specs=[
            pl.BlockSpec((1, 3, N), lambda b, t: (b, 0, 0)),
            pl.BlockSpec((1, _TILE, _K), lambda b, t: (b, t, 0)),
        ],
        out_specs=pl.BlockSpec((1, 8, 128), lambda b, t: (b, 0, 0)),
        out_shape=jax.ShapeDtypeStruct((B, 8, 128), jnp.float32),
        scratch_shapes=[pltpu.VMEM((_K, N), jnp.bfloat16)],
        compiler_params=pltpu.CompilerParams(
            dimension_semantics=("parallel", "arbitrary"),
        ),
    )(pmat, qmat)
    return (2.0 / (B * N)) * jnp.sum(out[:, 0, 0])


# speedup vs baseline: 1.3045x; 1.3045x over previous
import jax
import jax.numpy as jnp
from jax.experimental import pallas as pl
from jax.experimental.pallas import tpu as pltpu

_TILE = 256
_UNROLL = 4
_K = 8


def _chamfer_kernel(p_ref, q_ref, o_ref):
    n = p_ref.shape[2]
    p = p_ref[0]
    px = p[0:1, :]
    py = p[1:2, :]
    pz = p[2:3, :]
    pn = px * px + py * py + pz * pz
    hi = pn.astype(jnp.bfloat16)
    lo = (pn - hi.astype(jnp.float32)).astype(jnp.bfloat16)
    pb = jnp.concatenate(
        [
            (2.0 * px).astype(jnp.bfloat16),
            (-2.0 * py).astype(jnp.bfloat16),
            (-2.0 * pz).astype(jnp.bfloat16),
            hi,
            lo,
            jnp.zeros((_K - 5, n), jnp.bfloat16),
        ],
        axis=0,
    )

    def body(t, acc):
        for u in range(_UNROLL):
            base = (_UNROLL * t + u) * _TILE
            qb = q_ref[0, pl.ds(base, _TILE), :].astype(jnp.bfloat16)
            d = jnp.dot(qb, pb, preferred_element_type=jnp.float32)
            acc = acc + jnp.sum(jnp.min(d, axis=1))
        return acc

    acc = jax.lax.fori_loop(0, n // (_TILE * _UNROLL), body, jnp.float32(0.0))
    total = acc + jnp.sum(pn)
    o_ref[0] = jnp.full((8, 128), total, jnp.float32)


def kernel(xyz):
    B, N, _ = xyz.shape
    qmat = jnp.concatenate(
        [
            xyz,
            jnp.ones((B, N, 2), jnp.float32),
            jnp.zeros((B, N, _K - 5), jnp.float32),
        ],
        axis=2,
    )
    pmat = jnp.swapaxes(xyz, 1, 2)
    out = pl.pallas_call(
        _chamfer_kernel,
        grid=(B,),
        in_specs=[
            pl.BlockSpec((1, 3, N), lambda b: (b, 0, 0)),
            pl.BlockSpec((1, N, _K), lambda b: (b, 0, 0)),
        ],
        out_specs=pl.BlockSpec((1, 8, 128), lambda b: (b, 0, 0)),
        out_shape=jax.ShapeDtypeStruct((B, 8, 128), jnp.float32),
        compiler_params=pltpu.CompilerParams(
            dimension_semantics=("parallel",),
        ),
    )(pmat, qmat)
    return (2.0 / (B * N)) * jnp.sum(out[:, 0, 0])


# speedup vs baseline: 1.4035x; 1.0760x over previous
import jax
import jax.numpy as jnp
from jax.experimental import pallas as pl
from jax.experimental.pallas import tpu as pltpu

_TILE = 256
_UNROLL = 16
_K = 8


def _chamfer_kernel(p_ref, q_ref, o_ref):
    n = p_ref.shape[2]
    p = p_ref[0]
    px = p[0:1, :]
    py = p[1:2, :]
    pz = p[2:3, :]
    pn = px * px + py * py + pz * pz
    hi = pn.astype(jnp.bfloat16)
    lo = (pn - hi.astype(jnp.float32)).astype(jnp.bfloat16)
    pb = jnp.concatenate(
        [
            (2.0 * px).astype(jnp.bfloat16),
            (-2.0 * py).astype(jnp.bfloat16),
            (-2.0 * pz).astype(jnp.bfloat16),
            hi,
            lo,
            jnp.zeros((_K - 5, n), jnp.bfloat16),
        ],
        axis=0,
    )

    def body(t, acc):
        for u in range(_UNROLL):
            base = (_UNROLL * t + u) * _TILE
            qb = q_ref[0, pl.ds(base, _TILE), :].astype(jnp.bfloat16)
            d = jnp.dot(qb, pb, preferred_element_type=jnp.float32)
            acc = acc + jnp.sum(jnp.min(d, axis=1))
        return acc

    acc = jax.lax.fori_loop(0, n // (_TILE * _UNROLL), body, jnp.float32(0.0))
    total = acc + jnp.sum(pn)
    o_ref[0] = jnp.full((8, 128), total, jnp.float32)


def kernel(xyz):
    B, N, _ = xyz.shape
    qmat = jnp.concatenate(
        [
            xyz,
            jnp.ones((B, N, 2), jnp.float32),
            jnp.zeros((B, N, _K - 5), jnp.float32),
        ],
        axis=2,
    )
    pmat = jnp.swapaxes(xyz, 1, 2)
    out = pl.pallas_call(
        _chamfer_kernel,
        grid=(B,),
        in_specs=[
            pl.BlockSpec((1, 3, N), lambda b: (b, 0, 0)),
            pl.BlockSpec((1, N, _K), lambda b: (b, 0, 0)),
        ],
        out_specs=pl.BlockSpec((1, 8, 128), lambda b: (b, 0, 0)),
        out_shape=jax.ShapeDtypeStruct((B, 8, 128), jnp.float32),
        compiler_params=pltpu.CompilerParams(
            dimension_semantics=("parallel",),
        ),
    )(pmat, qmat)
    return (2.0 / (B * N)) * jnp.sum(out[:, 0, 0])


# speedup vs baseline: 1.6706x; 1.1903x over previous
import jax
import jax.numpy as jnp
from jax.experimental import pallas as pl
from jax.experimental.pallas import tpu as pltpu

_TILE = 256
_UNROLL = 16
_K = 8


def _chamfer_kernel(p_ref, q_ref, o_ref):
    n = p_ref.shape[2]
    p = p_ref[0]
    px = p[0:1, :]
    py = p[1:2, :]
    pz = p[2:3, :]
    pn = px * px + py * py + pz * pz
    hi = pn.astype(jnp.bfloat16)
    lo = (pn - hi.astype(jnp.float32)).astype(jnp.bfloat16)
    pb = jnp.concatenate(
        [
            (2.0 * px).astype(jnp.bfloat16),
            (-2.0 * py).astype(jnp.bfloat16),
            (-2.0 * pz).astype(jnp.bfloat16),
            hi,
            lo,
            jnp.zeros((_K - 5, n), jnp.bfloat16),
        ],
        axis=0,
    )

    nt = n // _TILE
    acc = jnp.float32(0.0)
    run_col = jnp.full((n,), jnp.inf, jnp.float32)
    for ti in range(nt):
        base = ti * _TILE
        q = q_ref[0, pl.ds(base, _TILE), :]
        qb = q.astype(jnp.bfloat16)
        d = jnp.dot(qb, pb[:, base:], preferred_element_type=jnp.float32)
        row_min = jnp.min(d, axis=1)
        dist = jnp.minimum(row_min, run_col[base:base + _TILE])
        acc = acc + jnp.sum(dist)
        if ti + 1 < nt:
            qx = q[:, 0:1]
            qy = q[:, 1:2]
            qz = q[:, 2:3]
            qn = qx * qx + qy * qy + qz * qz
            col_min = jnp.min(d + qn, axis=0) - pn[0, base:]
            upd = jnp.minimum(run_col[base:], col_min)
            run_col = upd if base == 0 else jnp.concatenate(
                [run_col[:base], upd])
    total = acc + jnp.sum(pn)
    o_ref[0] = jnp.full((8, 128), total, jnp.float32)


def kernel(xyz):
    B, N, _ = xyz.shape
    qmat = jnp.concatenate(
        [
            xyz,
            jnp.ones((B, N, 2), jnp.float32),
            jnp.zeros((B, N, _K - 5), jnp.float32),
        ],
        axis=2,
    )
    pmat = jnp.swapaxes(xyz, 1, 2)
    out = pl.pallas_call(
        _chamfer_kernel,
        grid=(B,),
        in_specs=[
            pl.BlockSpec((1, 3, N), lambda b: (b, 0, 0)),
            pl.BlockSpec((1, N, _K), lambda b: (b, 0, 0)),
        ],
        out_specs=pl.BlockSpec((1, 8, 128), lambda b: (b, 0, 0)),
        out_shape=jax.ShapeDtypeStruct((B, 8, 128), jnp.float32),
        compiler_params=pltpu.CompilerParams(
            dimension_semantics=("parallel",),
        ),
    )(pmat, qmat)
    return (2.0 / (B * N)) * jnp.sum(out[:, 0, 0])


# speedup vs baseline: 2.6065x; 1.5602x over previous
import jax
import jax.numpy as jnp
from jax.experimental import pallas as pl
from jax.experimental.pallas import tpu as pltpu

_TILE = 256
_K = 8


def _chamfer_kernel(p_ref, o_ref):
    n = p_ref.shape[2]
    p = p_ref[0]
    px = p[0:1, :]
    py = p[1:2, :]
    pz = p[2:3, :]
    pn = px * px + py * py + pz * pz
    hi = pn.astype(jnp.bfloat16)
    lo = (pn - hi.astype(jnp.float32)).astype(jnp.bfloat16)
    one = jnp.ones((1, n), jnp.bfloat16)
    zero = jnp.zeros((1, n), jnp.bfloat16)
    qb = jnp.concatenate(
        [px.astype(jnp.bfloat16), py.astype(jnp.bfloat16),
         pz.astype(jnp.bfloat16), one, one, hi, lo, zero], axis=0)
    pb = jnp.concatenate(
        [(2.0 * px).astype(jnp.bfloat16), (-2.0 * py).astype(jnp.bfloat16),
         (-2.0 * pz).astype(jnp.bfloat16), hi, lo, one, one, zero],
        axis=0)

    nt = n // _TILE
    acc = jnp.float32(0.0)
    run_col = jnp.full((n,), jnp.inf, jnp.float32)
    for ti in range(nt):
        base = ti * _TILE
        d = jax.lax.dot_general(
            qb[:, base:base + _TILE], pb[:, base:],
            (((0,), (0,)), ((), ())),
            preferred_element_type=jnp.float32)
        row_min = jnp.min(d, axis=1)
        dist = jnp.minimum(row_min, run_col[base:base + _TILE])
        acc = acc + jnp.sum(dist)
        if ti + 1 < nt:
            col_min = jnp.min(d, axis=0)
            upd = jnp.minimum(run_col[base:], col_min)
            run_col = upd if base == 0 else jnp.concatenate(
                [run_col[:base], upd])
    o_ref[0] = jnp.full((8, 128), acc, jnp.float32)


def kernel(xyz):
    B, N, _ = xyz.shape
    pmat = jnp.swapaxes(xyz, 1, 2)
    out = pl.pallas_call(
        _chamfer_kernel,
        grid=(B,),
        in_specs=[
            pl.BlockSpec((1, 3, N), lambda b: (b, 0, 0)),
        ],
        out_specs=pl.BlockSpec((1, 8, 128), lambda b: (b, 0, 0)),
        out_shape=jax.ShapeDtypeStruct((B, 8, 128), jnp.float32),
        compiler_params=pltpu.CompilerParams(
            dimension_semantics=("parallel",),
        ),
    )(pmat)
    return (2.0 / (B * N)) * jnp.sum(out[:, 0, 0])
